# baseline (speedup 1.0000x reference)
import jax
import jax.numpy as jnp
from jax.experimental import pallas as pl
from jax.experimental.pallas import tpu as pltpu

S_BLK = 1024


def _geno_block(x_ref, a_ref, p_ref, o_ref):
    p = p_ref[...]
    a = a_ref[...]
    x = x_ref[...]
    b = x.shape[0]
    for bi in range(b):
        y = jnp.dot(x[bi], a, preferred_element_type=jnp.float32)
        o_ref[bi] = y + p


def _tc_part(x, allele_embedding, position_embedding):
    B, S, N = x.shape
    D = allele_embedding.shape[1]
    grid = (S // S_BLK,)
    return pl.pallas_call(
        _geno_block,
        grid=grid,
        in_specs=[
            pl.BlockSpec((B, S_BLK, N), lambda i: (0, i, 0)),
            pl.BlockSpec((N, D), lambda i: (0, 0)),
            pl.BlockSpec((S_BLK, D), lambda i: (i, 0)),
        ],
        out_specs=pl.BlockSpec((B, S_BLK, D), lambda i: (0, i, 0)),
        out_shape=jax.ShapeDtypeStruct((B, S, D), jnp.float32),
    )(x, allele_embedding, position_embedding)


@jax.jit
def kernel(x, allele_embedding, position_embedding):
    out_a = _tc_part(x[:2], allele_embedding, position_embedding)
    out_b = _tc_part(x[2:], allele_embedding, position_embedding)
    return jnp.concatenate([out_a, out_b], axis=0)
